# baseline (speedup 1.0000x reference)
import functools
import math

import jax
import jax.numpy as jnp
from jax import lax
from jax.experimental import pallas as pl
from jax.experimental.pallas import tpu as pltpu
from jax.experimental.pallas import tpu_sc as plsc

VOCAB = 1000000
D_MODEL = 64
SCALE = math.sqrt(D_MODEL)

B, S = 4096, 200
N = B * S
NC, NS, L = 2, 16, 16
NW = NC * NS
N_PER_W = N // NW
C = 128
G = N_PER_W // C
H = G // 2

_mesh = plsc.VectorSubcoreMesh(core_axis_name="c", subcore_axis_name="s")


@functools.partial(
    pl.kernel,
    mesh=_mesh,
    compiler_params=pltpu.CompilerParams(use_tc_tiling_on_sc=True,
                                         needs_layout_passes=False),
    out_type=jax.ShapeDtypeStruct((N, D_MODEL), jnp.float32),
    scratch_types=[
        pltpu.VMEM((C,), jnp.int32),
        pltpu.VMEM((C,), jnp.int32),
        pltpu.VMEM((C,), jnp.int32),
        pltpu.VMEM((C,), jnp.int32),
        pltpu.VMEM((C, 2 * D_MODEL), jnp.float32),
        pltpu.VMEM((C, 2 * D_MODEL), jnp.float32),
        pltpu.VMEM((C, D_MODEL), jnp.float32),
        pltpu.VMEM((C, D_MODEL), jnp.float32),
        pltpu.SemaphoreType.DMA,
        pltpu.SemaphoreType.DMA,
        pltpu.SemaphoreType.DMA,
        pltpu.SemaphoreType.DMA,
    ],
)
def _emb_lookup(idx_hbm, table_hbm, out_hbm,
                raw0, raw1, pidx0, pidx1, pair0, pair1,
                row0, row1,
                gsem0, gsem1, osem0, osem1):
    wid = lax.axis_index("s") * NC + lax.axis_index("c")
    base = wid * N_PER_W

    raws = (raw0, raw1)
    pidxs = (pidx0, pidx1)
    pairs = (pair0, pair1)
    rows = (row0, row1)
    gsem = (gsem0, gsem1)
    osem = (osem0, osem1)

    def fetch_idx(g, b):
        pltpu.sync_copy(idx_hbm.at[pl.ds(base + g * C, C)], raws[b])

        def half(i, carry):
            sl = pl.ds(i * L, L)
            pidxs[b][sl] = jax.lax.shift_right_logical(raws[b][sl], 1)
            return carry

        lax.fori_loop(0, C // L, half, 0, unroll=4)

    def gather(b):
        pltpu.async_copy(table_hbm.at[pidxs[b]], pairs[b], gsem[b])

    def gather_wait(b):
        pltpu.make_async_copy(table_hbm.at[pidxs[b]], pairs[b],
                              gsem[b]).wait()

    def scatter(g, b):
        pltpu.async_copy(rows[b], out_hbm.at[pl.ds(base + g * C, C)], osem[b])

    def scatter_wait(b):
        pltpu.make_async_copy(rows[b], out_hbm.at[pl.ds(0, C)],
                              osem[b]).wait()

    def extract(b):
        src = pairs[b]
        dst = rows[b]
        lanes = lax.iota(jnp.int32, L)

        def group_body(m, carry):
            row_ids = m * L + lanes
            par = (raws[b][pl.ds(m * L, L)] & 1) * D_MODEL
            for c in range(D_MODEL):
                v = plsc.load_gather(src, [row_ids, par + c])
                plsc.store_scatter(dst, [row_ids, lanes * 0 + c], v)
            return carry

        lax.fori_loop(0, C // L, group_body, 0)

    fetch_idx(0, 0)
    gather(0)

    def step(h, carry):
        fetch_idx(2 * h + 1, 1)
        gather(1)
        gather_wait(0)

        @pl.when(h > 0)
        def _():
            scatter_wait(0)

        extract(0)
        scatter(2 * h, 0)

        @pl.when(h < H - 1)
        def _():
            fetch_idx(2 * h + 2, 0)
            gather(0)

        gather_wait(1)

        @pl.when(h > 0)
        def _():
            scatter_wait(1)

        extract(1)
        scatter(2 * h + 1, 1)
        return carry

    lax.fori_loop(0, H, step, 0)
    scatter_wait(0)
    scatter_wait(1)


def kernel(x, table):
    t_pairs = table.reshape(VOCAB // 2, 2 * D_MODEL) * jnp.float32(SCALE)
    out = _emb_lookup(x.reshape(N), t_pairs)
    return out.reshape(B, S, D_MODEL)


# speedup vs baseline: 2.0216x; 2.0216x over previous
import functools
import math

import jax
import jax.numpy as jnp
from jax import lax
from jax.experimental import pallas as pl
from jax.experimental.pallas import tpu as pltpu
from jax.experimental.pallas import tpu_sc as plsc

VOCAB = 1000000
D_MODEL = 64
SCALE = math.sqrt(D_MODEL)

B, S = 4096, 200
N = B * S
NC, NS, L = 2, 16, 16
NW = NC * NS
N_PER_W = N // NW
C = 128
G = N_PER_W // C
H = G // 2

_mesh = plsc.VectorSubcoreMesh(core_axis_name="c", subcore_axis_name="s")


@functools.partial(
    pl.kernel,
    mesh=_mesh,
    compiler_params=pltpu.CompilerParams(use_tc_tiling_on_sc=True,
                                         needs_layout_passes=False),
    out_type=jax.ShapeDtypeStruct((N, D_MODEL), jnp.float32),
    scratch_types=[
        pltpu.VMEM((C,), jnp.int32),
        pltpu.VMEM((C,), jnp.int32),
        pltpu.VMEM((C,), jnp.int32),
        pltpu.VMEM((C,), jnp.int32),
        pltpu.VMEM((C, 2 * D_MODEL), jnp.float32),
        pltpu.VMEM((C, 2 * D_MODEL), jnp.float32),
        pltpu.VMEM((C, D_MODEL), jnp.float32),
        pltpu.VMEM((C, D_MODEL), jnp.float32),
        pltpu.SemaphoreType.DMA,
        pltpu.SemaphoreType.DMA,
        pltpu.SemaphoreType.DMA,
        pltpu.SemaphoreType.DMA,
    ],
)
def _emb_lookup(idx_hbm, table_hbm, out_hbm,
                raw0, raw1, pidx0, pidx1, pair0, pair1,
                row0, row1,
                gsem0, gsem1, osem0, osem1):
    wid = lax.axis_index("s") * NC + lax.axis_index("c")
    base = wid * N_PER_W

    raws = (raw0, raw1)
    pidxs = (pidx0, pidx1)
    pairs = (pair0, pair1)
    rows = (row0, row1)
    gsem = (gsem0, gsem1)
    osem = (osem0, osem1)

    def fetch_idx(g, b):
        pltpu.sync_copy(idx_hbm.at[pl.ds(base + g * C, C)], raws[b])

        def half(i, carry):
            sl = pl.ds(i * L, L)
            pidxs[b][sl] = jax.lax.shift_right_logical(raws[b][sl], 1)
            return carry

        lax.fori_loop(0, C // L, half, 0, unroll=4)

    def gather(b):
        pltpu.async_copy(table_hbm.at[pidxs[b]], pairs[b], gsem[b])

    def gather_wait(b):
        pltpu.make_async_copy(table_hbm.at[pidxs[b]], pairs[b],
                              gsem[b]).wait()

    def scatter(g, b):
        pltpu.async_copy(rows[b], out_hbm.at[pl.ds(base + g * C, C)], osem[b])

    def scatter_wait(b):
        pltpu.make_async_copy(rows[b], out_hbm.at[pl.ds(0, C)],
                              osem[b]).wait()

    def extract(b):
        src = pairs[b]
        dst = rows[b]

        def group_body(m, carry):
            par = (raws[b][pl.ds(m * L, L)] & 1) * D_MODEL
            for l in range(L):
                r = m * L + l
                off = par[l]
                for j in range(D_MODEL // L):
                    dst[r, pl.ds(j * L, L)] = src[r, pl.ds(off + j * L, L)]
            return carry

        lax.fori_loop(0, C // L, group_body, 0)

    fetch_idx(0, 0)
    gather(0)

    def step(h, carry):
        fetch_idx(2 * h + 1, 1)
        gather(1)
        gather_wait(0)

        @pl.when(h > 0)
        def _():
            scatter_wait(0)

        extract(0)
        scatter(2 * h, 0)

        @pl.when(h < H - 1)
        def _():
            fetch_idx(2 * h + 2, 0)
            gather(0)

        gather_wait(1)

        @pl.when(h > 0)
        def _():
            scatter_wait(1)

        extract(1)
        scatter(2 * h + 1, 1)
        return carry

    lax.fori_loop(0, H, step, 0)
    scatter_wait(0)
    scatter_wait(1)


def kernel(x, table):
    t_pairs = table.reshape(VOCAB // 2, 2 * D_MODEL) * jnp.float32(SCALE)
    out = _emb_lookup(x.reshape(N), t_pairs)
    return out.reshape(B, S, D_MODEL)


# speedup vs baseline: 2.2250x; 1.1006x over previous
import functools
import math

import jax
import jax.numpy as jnp
from jax import lax
from jax.experimental import pallas as pl
from jax.experimental.pallas import tpu as pltpu
from jax.experimental.pallas import tpu_sc as plsc

VOCAB = 1000000
D_MODEL = 64
SCALE = math.sqrt(D_MODEL)

B, S = 4096, 200
N = B * S
NC, NS, L = 2, 16, 16
NW = NC * NS
N_PER_W = N // NW
C = 128
G = N_PER_W // C
H = G // 2

_mesh = plsc.VectorSubcoreMesh(core_axis_name="c", subcore_axis_name="s")


@functools.partial(
    pl.kernel,
    mesh=_mesh,
    compiler_params=pltpu.CompilerParams(use_tc_tiling_on_sc=False),
    out_type=jax.ShapeDtypeStruct((N, D_MODEL), jnp.float32),
    scratch_types=[
        pltpu.VMEM((C,), jnp.int32),
        pltpu.VMEM((C,), jnp.int32),
        pltpu.VMEM((C, D_MODEL), jnp.float32),
        pltpu.VMEM((C, D_MODEL), jnp.float32),
        pltpu.SemaphoreType.DMA,
        pltpu.SemaphoreType.DMA,
        pltpu.SemaphoreType.DMA,
        pltpu.SemaphoreType.DMA,
    ],
)
def _emb_lookup(idx_hbm, table_hbm, out_hbm,
                idx0, idx1, row0, row1,
                gsem0, gsem1, osem0, osem1):
    wid = lax.axis_index("s") * NC + lax.axis_index("c")
    base = wid * N_PER_W

    idxs = (idx0, idx1)
    rows = (row0, row1)
    gsem = (gsem0, gsem1)
    osem = (osem0, osem1)

    def fetch_idx(g, b):
        pltpu.sync_copy(idx_hbm.at[pl.ds(base + g * C, C)], idxs[b])

    def gather(b):
        pltpu.async_copy(table_hbm.at[idxs[b]], rows[b], gsem[b])

    def gather_wait(b):
        pltpu.make_async_copy(table_hbm.at[idxs[b]], rows[b],
                              gsem[b]).wait()

    def scatter(g, b):
        pltpu.async_copy(rows[b], out_hbm.at[pl.ds(base + g * C, C)], osem[b])

    def scatter_wait(b):
        pltpu.make_async_copy(rows[b], out_hbm.at[pl.ds(0, C)],
                              osem[b]).wait()

    def scale(b):
        buf = rows[b]

        def row_body(r, carry):
            for j in range(D_MODEL // L):
                sl = pl.ds(j * L, L)
                buf[r, sl] = buf[r, sl] * SCALE
            return carry

        lax.fori_loop(0, C, row_body, 0, unroll=4)

    fetch_idx(0, 0)
    gather(0)

    def step(h, carry):
        fetch_idx(2 * h + 1, 1)
        gather(1)
        gather_wait(0)

        @pl.when(h > 0)
        def _():
            scatter_wait(0)

        scale(0)
        scatter(2 * h, 0)

        @pl.when(h < H - 1)
        def _():
            fetch_idx(2 * h + 2, 0)
            gather(0)

        gather_wait(1)

        @pl.when(h > 0)
        def _():
            scatter_wait(1)

        scale(1)
        scatter(2 * h + 1, 1)
        return carry

    lax.fori_loop(0, H, step, 0)
    scatter_wait(0)
    scatter_wait(1)


def kernel(x, table):
    out = _emb_lookup(x.reshape(N), table)
    return out.reshape(B, S, D_MODEL)


# speedup vs baseline: 2.3468x; 1.0547x over previous
import functools
import math

import jax
import jax.numpy as jnp
from jax import lax
from jax.experimental import pallas as pl
from jax.experimental.pallas import tpu as pltpu
from jax.experimental.pallas import tpu_sc as plsc

VOCAB = 1000000
D_MODEL = 64
SCALE = math.sqrt(D_MODEL)

B, S = 4096, 200
N = B * S
NC, NS, L = 2, 16, 16
NW = NC * NS
N_PER_W = N // NW
C = 256
G = N_PER_W // C
H = G // 2

_mesh = plsc.VectorSubcoreMesh(core_axis_name="c", subcore_axis_name="s")


@functools.partial(
    pl.kernel,
    mesh=_mesh,
    compiler_params=pltpu.CompilerParams(use_tc_tiling_on_sc=False),
    out_type=jax.ShapeDtypeStruct((N, D_MODEL), jnp.float32),
    scratch_types=[
        pltpu.VMEM((N_PER_W,), jnp.int32),
        pltpu.VMEM((C, D_MODEL), jnp.float32),
        pltpu.VMEM((C, D_MODEL), jnp.float32),
        pltpu.SemaphoreType.DMA,
        pltpu.SemaphoreType.DMA,
        pltpu.SemaphoreType.DMA,
        pltpu.SemaphoreType.DMA,
    ],
)
def _emb_lookup(idx_hbm, table_hbm, out_hbm,
                idx_v, row0, row1,
                gsem0, gsem1, osem0, osem1):
    wid = lax.axis_index("s") * NC + lax.axis_index("c")
    base = wid * N_PER_W

    pltpu.sync_copy(idx_hbm.at[pl.ds(base, N_PER_W)], idx_v)

    rows = (row0, row1)
    gsem = (gsem0, gsem1)
    osem = (osem0, osem1)

    def gather(g, b):
        pltpu.async_copy(table_hbm.at[idx_v.at[pl.ds(g * C, C)]],
                         rows[b], gsem[b])

    def gather_wait(g, b):
        pltpu.make_async_copy(table_hbm.at[idx_v.at[pl.ds(g * C, C)]],
                              rows[b], gsem[b]).wait()

    def scatter(g, b):
        pltpu.async_copy(rows[b], out_hbm.at[pl.ds(base + g * C, C)], osem[b])

    def scatter_wait(b):
        pltpu.make_async_copy(rows[b], out_hbm.at[pl.ds(0, C)],
                              osem[b]).wait()

    def scale(b):
        buf = rows[b]

        def row_body(r, carry):
            for j in range(D_MODEL // L):
                sl = pl.ds(j * L, L)
                buf[r, sl] = buf[r, sl] * SCALE
            return carry

        lax.fori_loop(0, C, row_body, 0, unroll=4)

    gather(0, 0)

    def step(h, carry):
        gather(2 * h + 1, 1)
        gather_wait(2 * h, 0)

        @pl.when(h > 0)
        def _():
            scatter_wait(0)

        scale(0)
        scatter(2 * h, 0)

        @pl.when(h < H - 1)
        def _():
            gather(2 * h + 2, 0)

        gather_wait(2 * h + 1, 1)

        @pl.when(h > 0)
        def _():
            scatter_wait(1)

        scale(1)
        scatter(2 * h + 1, 1)
        return carry

    lax.fori_loop(0, H, step, 0)
    scatter_wait(0)
    scatter_wait(1)


def kernel(x, table):
    out = _emb_lookup(x.reshape(N), table)
    return out.reshape(B, S, D_MODEL)
